# baseline (speedup 1.0000x reference)
import functools

import jax
import jax.numpy as jnp
from jax import lax
from jax.experimental import pallas as pl
from jax.experimental.pallas import tpu as pltpu
from jax.experimental.pallas import tpu_sc as plsc

N = 10000
E = 320000
D = 128
H = 128
G = 64
C = 5

NC = 2
NS = 16
NW = NC * NS

CHUNK = 128
CPT = 80
EPT = CPT * CHUNK
EPAD = NW * EPT

AGG_ROWS = 10112
RPT = AGG_ROWS // NS
SPT = N // NS

FW = H // 2

_MESH = plsc.VectorSubcoreMesh(core_axis_name="c", subcore_axis_name="s")

ROW_BLK = 1000
NBLK = N // ROW_BLK


CW = 16


@functools.partial(
    pl.kernel,
    out_type=jax.ShapeDtypeStruct((NC, AGG_ROWS, CW), jnp.float32),
    mesh=_MESH,
    scratch_types=[
        pltpu.VMEM((CPT, CHUNK), jnp.int32),
        pltpu.VMEM((CHUNK, CW), jnp.float32),
        pltpu.VMEM((CHUNK, CW), jnp.float32),
        pltpu.VMEM_SHARED((AGG_ROWS, CW), jnp.float32),
    ],
    compiler_params=pltpu.CompilerParams(use_tc_tiling_on_sc=False),
)
def _sc_hist(col_hbm, deg_out, colv, onesv, zerov, ctab):
    cid = lax.axis_index("c")
    sid = lax.axis_index("s")
    wid = cid * NS + sid

    pltpu.sync_copy(col_hbm.at[wid], colv)

    zeros16 = jnp.zeros((16,), jnp.float32)
    ones16 = jnp.ones((16,), jnp.float32)

    def _fill(t, carry):
        onesv[t, :] = ones16
        zerov[t, :] = zeros16
        return carry

    lax.fori_loop(0, CHUNK, _fill, 0)

    base = sid * RPT
    for t in range(4):
        pltpu.sync_copy(zerov, ctab.at[pl.ds(base + t * CHUNK, CHUNK)])
    pltpu.sync_copy(zerov.at[pl.ds(0, RPT - 4 * CHUNK)],
                    ctab.at[pl.ds(base + 4 * CHUNK, RPT - 4 * CHUNK)])
    plsc.subcore_barrier()

    def _count(j, carry):
        pltpu.sync_copy(onesv, ctab.at[colv.at[j]], add=True)
        return carry

    lax.fori_loop(0, CPT, _count, 0)
    plsc.subcore_barrier()

    pltpu.sync_copy(ctab.at[pl.ds(base, RPT)],
                    deg_out.at[cid, pl.ds(base, RPT)])


def _mm_body(x_ref, w_ref, d0_ref, d1_ref, out0_ref, out1_ref):
    dinv = lax.rsqrt(d0_ref[...] + d1_ref[...] + 1.0)
    hp = jnp.dot(x_ref[...], w_ref[...],
                 preferred_element_type=jnp.float32) * dinv
    out0_ref[...] = hp[:, :FW]
    out1_ref[...] = hp[:, FW:]


def _mm(x, W1, deg0, deg1):
    return pl.pallas_call(
        _mm_body,
        grid=(NBLK,),
        in_specs=[
            pl.BlockSpec((ROW_BLK, D), lambda i: (i, 0)),
            pl.BlockSpec((D, H), lambda i: (0, 0)),
            pl.BlockSpec((ROW_BLK, 1), lambda i: (i, 0)),
            pl.BlockSpec((ROW_BLK, 1), lambda i: (i, 0)),
        ],
        out_specs=[
            pl.BlockSpec((ROW_BLK, FW), lambda i: (i, 0)),
            pl.BlockSpec((ROW_BLK, FW), lambda i: (i, 0)),
        ],
        out_shape=[
            jax.ShapeDtypeStruct((N, FW), jnp.float32),
            jax.ShapeDtypeStruct((N, FW), jnp.float32),
        ],
    )(x, W1, deg0, deg1)


IH = CPT // 2


@functools.partial(
    pl.kernel,
    out_type=jax.ShapeDtypeStruct((2, NC, AGG_ROWS, FW), jnp.float32),
    mesh=_MESH,
    scratch_types=[
        pltpu.VMEM((IH, CHUNK), jnp.int32),
        pltpu.VMEM((IH, CHUNK), jnp.int32),
        pltpu.VMEM((CHUNK, FW), jnp.float32),
        pltpu.VMEM((CHUNK, FW), jnp.float32),
        pltpu.VMEM_SHARED((N, FW), jnp.float32),
        pltpu.VMEM_SHARED((AGG_ROWS, FW), jnp.float32),
        pltpu.SemaphoreType.DMA,
        pltpu.SemaphoreType.DMA,
        pltpu.SemaphoreType.DMA,
        pltpu.SemaphoreType.DMA,
    ],
    compiler_params=pltpu.CompilerParams(use_tc_tiling_on_sc=False),
)
def _sc_agg(hp0_hbm, hp1_hbm, row_hbm, col_hbm, agg_out,
            rowv, colv, buf0, buf1, stab, sacc, sem0, sem1, ssem0, ssem1):
    cid = lax.axis_index("c")
    sid = lax.axis_index("s")
    wid = cid * NS + sid

    zeros16 = jnp.zeros((16,), jnp.float32)

    def _zero(t, carry):
        buf0[t >> 2, pl.ds((t & 3) * 16, 16)] = zeros16
        return carry

    lax.fori_loop(0, CHUNK * 4, _zero, 0)

    base = sid * RPT
    for fhalf in range(2):
        hp_hbm = hp0_hbm if fhalf == 0 else hp1_hbm
        pltpu.sync_copy(hp_hbm.at[pl.ds(sid * SPT, SPT)],
                        stab.at[pl.ds(sid * SPT, SPT)])
        for t in range(4):
            pltpu.sync_copy(buf0, sacc.at[pl.ds(base + t * CHUNK, CHUNK)])
        pltpu.sync_copy(buf0.at[pl.ds(0, RPT - 4 * CHUNK)],
                        sacc.at[pl.ds(base + 4 * CHUNK, RPT - 4 * CHUNK)])
        plsc.subcore_barrier()

        for ihalf in range(2):
            off = ihalf * IH
            pltpu.sync_copy(row_hbm.at[wid, pl.ds(off, IH)], rowv)
            pltpu.sync_copy(col_hbm.at[wid, pl.ds(off, IH)], colv)

            pltpu.async_copy(stab.at[rowv.at[0]], buf0, sem0)
            pltpu.async_copy(stab.at[rowv.at[1]], buf1, sem1)

            def _step(jj, carry):
                j = jj * 2
                pltpu.make_async_copy(stab.at[rowv.at[j]], buf0, sem0).wait()
                pltpu.async_copy(buf0, sacc.at[colv.at[j]], ssem0, add=True)
                pltpu.make_async_copy(
                    stab.at[rowv.at[j + 1]], buf1, sem1).wait()
                pltpu.async_copy(buf1, sacc.at[colv.at[j + 1]], ssem1,
                                 add=True)
                pltpu.make_async_copy(buf0, sacc.at[colv.at[j]],
                                      ssem0).wait()
                pltpu.async_copy(stab.at[rowv.at[j + 2]], buf0, sem0)
                pltpu.make_async_copy(buf1, sacc.at[colv.at[j + 1]],
                                      ssem1).wait()
                pltpu.async_copy(stab.at[rowv.at[j + 3]], buf1, sem1)
                return carry

            lax.fori_loop(0, IH // 2 - 2, _step, 0)

            for j in (IH - 4, IH - 3):
                buf, sem = (buf0, sem0) if j % 2 == 0 else (buf1, sem1)
                pltpu.make_async_copy(stab.at[rowv.at[j]], buf, sem).wait()
                pltpu.sync_copy(buf, sacc.at[colv.at[j]], add=True)
                pltpu.async_copy(stab.at[rowv.at[j + 2]], buf, sem)
            for j in (IH - 2, IH - 1):
                buf, sem = (buf0, sem0) if j % 2 == 0 else (buf1, sem1)
                pltpu.make_async_copy(stab.at[rowv.at[j]], buf, sem).wait()
                pltpu.sync_copy(buf, sacc.at[colv.at[j]], add=True)

        plsc.subcore_barrier()
        pltpu.sync_copy(sacc.at[pl.ds(sid * RPT, RPT)],
                        agg_out.at[fhalf, cid, pl.ds(sid * RPT, RPT)])
        if fhalf == 0:
            lax.fori_loop(0, CHUNK * 4, _zero, 0)


def _epi_body(p00_ref, p01_ref, p10_ref, p11_ref, hp0_ref, hp1_ref,
              d0_ref, d1_ref, b_ref, b1_ref, w2_ref, b2_ref, out_ref,
              sums, counts):
    i = pl.program_id(0)
    dinv = lax.rsqrt(d0_ref[...] + d1_ref[...] + 1.0)
    pl_half = (jnp.reshape(p00_ref[...], (ROW_BLK, FW))
               + jnp.reshape(p01_ref[...], (ROW_BLK, FW)) + hp0_ref[...])
    pr_half = (jnp.reshape(p10_ref[...], (ROW_BLK, FW))
               + jnp.reshape(p11_ref[...], (ROW_BLK, FW)) + hp1_ref[...])
    agg = jnp.concatenate([pl_half, pr_half], axis=1) * dinv + b1_ref[...]
    h2 = jnp.maximum(agg, 0.0)
    bm = jnp.reshape(b_ref[...], (1, ROW_BLK))
    seg = lax.broadcasted_iota(jnp.int32, (G, ROW_BLK), 0)
    M = (bm == seg).astype(jnp.float32)
    part = jnp.dot(M, h2, preferred_element_type=jnp.float32)
    cpart = jnp.dot(M, jnp.ones((ROW_BLK, H), jnp.float32),
                    preferred_element_type=jnp.float32)

    @pl.when(i == 0)
    def _():
        sums[...] = part
        counts[...] = cpart

    @pl.when(i > 0)
    def _():
        sums[...] += part
        counts[...] += cpart

    @pl.when(i == NBLK - 1)
    def _():
        pooled = sums[...] / jnp.maximum(counts[...], 1.0)
        logits = jnp.dot(pooled, w2_ref[...],
                         preferred_element_type=jnp.float32) + b2_ref[...]
        m = jnp.max(logits, axis=1, keepdims=True)
        lse = jnp.log(jnp.sum(jnp.exp(logits - m), axis=1, keepdims=True))
        out_ref[...] = logits - m - lse


def _epi(aggp, hp0, hp1, deg0, deg1, batch3, b1r, W2p, b2p):
    return pl.pallas_call(
        _epi_body,
        grid=(NBLK,),
        in_specs=[
            pl.BlockSpec((1, 1, ROW_BLK, FW), lambda i: (0, 0, i, 0)),
            pl.BlockSpec((1, 1, ROW_BLK, FW), lambda i: (0, 1, i, 0)),
            pl.BlockSpec((1, 1, ROW_BLK, FW), lambda i: (1, 0, i, 0)),
            pl.BlockSpec((1, 1, ROW_BLK, FW), lambda i: (1, 1, i, 0)),
            pl.BlockSpec((ROW_BLK, FW), lambda i: (i, 0)),
            pl.BlockSpec((ROW_BLK, FW), lambda i: (i, 0)),
            pl.BlockSpec((ROW_BLK, 1), lambda i: (i, 0)),
            pl.BlockSpec((ROW_BLK, 1), lambda i: (i, 0)),
            pl.BlockSpec((1, 1, ROW_BLK), lambda i: (i, 0, 0)),
            pl.BlockSpec((1, H), lambda i: (0, 0)),
            pl.BlockSpec((H, 128), lambda i: (0, 0)),
            pl.BlockSpec((1, 128), lambda i: (0, 0)),
        ],
        out_specs=pl.BlockSpec((G, 128), lambda i: (0, 0)),
        out_shape=jax.ShapeDtypeStruct((G, 128), jnp.float32),
        scratch_shapes=[
            pltpu.VMEM((G, 128), jnp.float32),
            pltpu.VMEM((G, 128), jnp.float32),
        ],
    )(aggp, aggp, aggp, aggp, hp0, hp1, deg0, deg1, batch3, b1r, W2p, b2p)


def kernel(x, edge_index, batch, W1, b1, W2, b2):
    row = edge_index[0]
    col = edge_index[1]
    pad = EPAD - E
    row_p = jnp.concatenate(
        [row, jnp.zeros((pad,), jnp.int32)]).reshape(NW, CPT, CHUNK)
    col_p = jnp.concatenate(
        [col, jnp.full((pad,), N, jnp.int32)]).reshape(NW, CPT, CHUNK)

    degp = _sc_hist(col_p)
    deg0 = degp[0, :N, 0].reshape(N, 1)
    deg1 = degp[1, :N, 0].reshape(N, 1)

    hp0, hp1 = _mm(x, W1, deg0, deg1)

    aggp = _sc_agg(hp0, hp1, row_p, col_p)

    batch3 = batch.reshape(NBLK, 1, ROW_BLK)
    b1r = b1.reshape(1, H)
    W2p = jnp.pad(W2, ((0, 0), (0, 128 - C)))
    b2p = jnp.concatenate(
        [b2, jnp.full((128 - C,), -1e30, jnp.float32)]).reshape(1, 128)

    outp = _epi(aggp, hp0, hp1, deg0, deg1, batch3, b1r, W2p, b2p)
    return outp[:, :C]


# speedup vs baseline: 1.4626x; 1.4626x over previous
import functools

import jax
import jax.numpy as jnp
from jax import lax
from jax.experimental import pallas as pl
from jax.experimental.pallas import tpu as pltpu
from jax.experimental.pallas import tpu_sc as plsc

N = 10000
E = 320000
D = 128
H = 128
G = 64
C = 5

NC = 2
NS = 16
NW = NC * NS

CHUNK = 128
CPT = 80
EPT = CPT * CHUNK
EPAD = NW * EPT

AGG_ROWS = 10112
RPT = AGG_ROWS // NS
SPT = N // NS

_MESH = plsc.VectorSubcoreMesh(core_axis_name="c", subcore_axis_name="s")

ROW_BLK = 1000
NBLK = N // ROW_BLK


CW = 16


@functools.partial(
    pl.kernel,
    out_type=jax.ShapeDtypeStruct((NC, AGG_ROWS, CW), jnp.float32),
    mesh=_MESH,
    scratch_types=[
        pltpu.VMEM((CPT, CHUNK), jnp.int32),
        pltpu.VMEM((CHUNK, CW), jnp.float32),
        pltpu.VMEM((CHUNK, CW), jnp.float32),
        pltpu.VMEM_SHARED((AGG_ROWS, CW), jnp.float32),
    ],
    compiler_params=pltpu.CompilerParams(use_tc_tiling_on_sc=False),
)
def _sc_hist(col_hbm, deg_out, colv, onesv, zerov, ctab):
    cid = lax.axis_index("c")
    sid = lax.axis_index("s")
    wid = cid * NS + sid

    pltpu.sync_copy(col_hbm.at[wid], colv)

    zeros16 = jnp.zeros((16,), jnp.float32)
    ones16 = jnp.ones((16,), jnp.float32)

    def _fill(t, carry):
        onesv[t, :] = ones16
        zerov[t, :] = zeros16
        return carry

    lax.fori_loop(0, CHUNK, _fill, 0)

    base = sid * RPT
    for t in range(4):
        pltpu.sync_copy(zerov, ctab.at[pl.ds(base + t * CHUNK, CHUNK)])
    pltpu.sync_copy(zerov.at[pl.ds(0, RPT - 4 * CHUNK)],
                    ctab.at[pl.ds(base + 4 * CHUNK, RPT - 4 * CHUNK)])
    plsc.subcore_barrier()

    def _count(j, carry):
        pltpu.sync_copy(onesv, ctab.at[colv.at[j]], add=True)
        return carry

    lax.fori_loop(0, CPT, _count, 0)
    plsc.subcore_barrier()

    pltpu.sync_copy(ctab.at[pl.ds(base, RPT)],
                    deg_out.at[cid, pl.ds(base, RPT)])


def _mm_body(x_ref, w_ref, d0_ref, d1_ref, out_ref):
    dinv = lax.rsqrt(d0_ref[...] + d1_ref[...] + 1.0)
    hp = jnp.dot(x_ref[...], w_ref[...],
                 preferred_element_type=jnp.float32) * dinv
    out_ref[...] = hp.astype(jnp.bfloat16)


def _mm(x, W1, deg0, deg1):
    return pl.pallas_call(
        _mm_body,
        grid=(NBLK,),
        in_specs=[
            pl.BlockSpec((ROW_BLK, D), lambda i: (i, 0)),
            pl.BlockSpec((D, H), lambda i: (0, 0)),
            pl.BlockSpec((ROW_BLK, 1), lambda i: (i, 0)),
            pl.BlockSpec((ROW_BLK, 1), lambda i: (i, 0)),
        ],
        out_specs=pl.BlockSpec((ROW_BLK, H), lambda i: (i, 0)),
        out_shape=jax.ShapeDtypeStruct((N, H), jnp.bfloat16),
    )(x, W1, deg0, deg1)


IH = CPT // 2


@functools.partial(
    pl.kernel,
    out_type=jax.ShapeDtypeStruct((NC, AGG_ROWS, H), jnp.bfloat16),
    mesh=_MESH,
    scratch_types=[
        pltpu.VMEM((IH, CHUNK), jnp.int32),
        pltpu.VMEM((IH, CHUNK), jnp.int32),
        pltpu.VMEM((CHUNK, H), jnp.bfloat16),
        pltpu.VMEM((CHUNK, H), jnp.bfloat16),
        pltpu.VMEM_SHARED((N, H), jnp.bfloat16),
        pltpu.VMEM_SHARED((AGG_ROWS, H), jnp.bfloat16),
        pltpu.SemaphoreType.DMA,
        pltpu.SemaphoreType.DMA,
    ],
    compiler_params=pltpu.CompilerParams(use_tc_tiling_on_sc=False),
)
def _sc_agg(hp_hbm, row_hbm, col_hbm, agg_out,
            rowv, colv, buf0, buf1, stab, sacc, sem0, sem1):
    cid = lax.axis_index("c")
    sid = lax.axis_index("s")
    wid = cid * NS + sid

    pltpu.sync_copy(hp_hbm.at[pl.ds(sid * SPT, SPT)],
                    stab.at[pl.ds(sid * SPT, SPT)])

    zeros32 = jnp.zeros((32,), jnp.bfloat16)

    def _zero(t, carry):
        buf0[t >> 2, pl.ds((t & 3) * 32, 32)] = zeros32
        return carry

    lax.fori_loop(0, CHUNK * 4, _zero, 0)

    base = sid * RPT
    for t in range(4):
        pltpu.sync_copy(buf0, sacc.at[pl.ds(base + t * CHUNK, CHUNK)])
    pltpu.sync_copy(buf0.at[pl.ds(0, RPT - 4 * CHUNK)],
                    sacc.at[pl.ds(base + 4 * CHUNK, RPT - 4 * CHUNK)])
    plsc.subcore_barrier()

    for ihalf in range(2):
        off = ihalf * IH
        pltpu.sync_copy(row_hbm.at[wid, pl.ds(off, IH)], rowv)
        pltpu.sync_copy(col_hbm.at[wid, pl.ds(off, IH)], colv)

        pltpu.async_copy(stab.at[rowv.at[0]], buf0, sem0)
        pltpu.async_copy(stab.at[rowv.at[1]], buf1, sem1)

        def _step(jj, carry):
            j = jj * 2
            pltpu.make_async_copy(stab.at[rowv.at[j]], buf0, sem0).wait()
            pltpu.sync_copy(buf0, sacc.at[colv.at[j]], add=True)
            pltpu.async_copy(stab.at[rowv.at[j + 2]], buf0, sem0)
            pltpu.make_async_copy(
                stab.at[rowv.at[j + 1]], buf1, sem1).wait()
            pltpu.sync_copy(buf1, sacc.at[colv.at[j + 1]], add=True)
            pltpu.async_copy(stab.at[rowv.at[j + 3]], buf1, sem1)
            return carry

        lax.fori_loop(0, IH // 2 - 2, _step, 0)

        for j in (IH - 4, IH - 3):
            buf, sem = (buf0, sem0) if j % 2 == 0 else (buf1, sem1)
            pltpu.make_async_copy(stab.at[rowv.at[j]], buf, sem).wait()
            pltpu.sync_copy(buf, sacc.at[colv.at[j]], add=True)
            pltpu.async_copy(stab.at[rowv.at[j + 2]], buf, sem)
        for j in (IH - 2, IH - 1):
            buf, sem = (buf0, sem0) if j % 2 == 0 else (buf1, sem1)
            pltpu.make_async_copy(stab.at[rowv.at[j]], buf, sem).wait()
            pltpu.sync_copy(buf, sacc.at[colv.at[j]], add=True)

    plsc.subcore_barrier()
    pltpu.sync_copy(sacc.at[pl.ds(sid * RPT, RPT)],
                    agg_out.at[cid, pl.ds(sid * RPT, RPT)])


def _epi_body(p0_ref, p1_ref, hp_ref, d0_ref, d1_ref, b_ref, b1_ref,
              w2_ref, b2_ref, out_ref, sums, counts):
    i = pl.program_id(0)
    dinv = lax.rsqrt(d0_ref[...] + d1_ref[...] + 1.0)
    p0 = jnp.reshape(p0_ref[...], (ROW_BLK, H)).astype(jnp.float32)
    p1 = jnp.reshape(p1_ref[...], (ROW_BLK, H)).astype(jnp.float32)
    hp = hp_ref[...].astype(jnp.float32)
    agg = (p0 + p1 + hp) * dinv + b1_ref[...]
    h2 = jnp.maximum(agg, 0.0)
    bm = jnp.reshape(b_ref[...], (1, ROW_BLK))
    seg = lax.broadcasted_iota(jnp.int32, (G, ROW_BLK), 0)
    M = (bm == seg).astype(jnp.float32)
    part = jnp.dot(M, h2, preferred_element_type=jnp.float32)
    cpart = jnp.dot(M, jnp.ones((ROW_BLK, H), jnp.float32),
                    preferred_element_type=jnp.float32)

    @pl.when(i == 0)
    def _():
        sums[...] = part
        counts[...] = cpart

    @pl.when(i > 0)
    def _():
        sums[...] += part
        counts[...] += cpart

    @pl.when(i == NBLK - 1)
    def _():
        pooled = sums[...] / jnp.maximum(counts[...], 1.0)
        logits = jnp.dot(pooled, w2_ref[...],
                         preferred_element_type=jnp.float32) + b2_ref[...]
        m = jnp.max(logits, axis=1, keepdims=True)
        lse = jnp.log(jnp.sum(jnp.exp(logits - m), axis=1, keepdims=True))
        out_ref[...] = logits - m - lse


def _epi(aggp, hp, deg0, deg1, batch3, b1r, W2p, b2p):
    return pl.pallas_call(
        _epi_body,
        grid=(NBLK,),
        in_specs=[
            pl.BlockSpec((1, ROW_BLK, H), lambda i: (0, i, 0)),
            pl.BlockSpec((1, ROW_BLK, H), lambda i: (1, i, 0)),
            pl.BlockSpec((ROW_BLK, H), lambda i: (i, 0)),
            pl.BlockSpec((ROW_BLK, 1), lambda i: (i, 0)),
            pl.BlockSpec((ROW_BLK, 1), lambda i: (i, 0)),
            pl.BlockSpec((1, 1, ROW_BLK), lambda i: (i, 0, 0)),
            pl.BlockSpec((1, H), lambda i: (0, 0)),
            pl.BlockSpec((H, 128), lambda i: (0, 0)),
            pl.BlockSpec((1, 128), lambda i: (0, 0)),
        ],
        out_specs=pl.BlockSpec((G, 128), lambda i: (0, 0)),
        out_shape=jax.ShapeDtypeStruct((G, 128), jnp.float32),
        scratch_shapes=[
            pltpu.VMEM((G, 128), jnp.float32),
            pltpu.VMEM((G, 128), jnp.float32),
        ],
    )(aggp, aggp, hp, deg0, deg1, batch3, b1r, W2p, b2p)


def kernel(x, edge_index, batch, W1, b1, W2, b2):
    row = edge_index[0]
    col = edge_index[1]
    pad = EPAD - E
    row_p = jnp.concatenate(
        [row, jnp.zeros((pad,), jnp.int32)]).reshape(NW, CPT, CHUNK)
    col_p = jnp.concatenate(
        [col, jnp.full((pad,), N, jnp.int32)]).reshape(NW, CPT, CHUNK)

    degp = _sc_hist(col_p)
    deg0 = degp[0, :N, 0].reshape(N, 1)
    deg1 = degp[1, :N, 0].reshape(N, 1)

    hp = _mm(x, W1, deg0, deg1)

    aggp = _sc_agg(hp, row_p, col_p)

    batch3 = batch.reshape(NBLK, 1, ROW_BLK)
    b1r = b1.reshape(1, H)
    W2p = jnp.pad(W2, ((0, 0), (0, 128 - C)))
    b2p = jnp.concatenate(
        [b2, jnp.full((128 - C,), -1e30, jnp.float32)]).reshape(1, 128)

    outp = _epi(aggp, hp, deg0, deg1, batch3, b1r, W2p, b2p)
    return outp[:, :C]
